# baseline (speedup 1.0000x reference)
import functools

import jax
import jax.numpy as jnp
from jax.experimental import pallas as pl
from jax.experimental.pallas import tpu as pltpu


def _half(adj, feats, feats_blk, wl1, wc):
    acc = jnp.dot(adj, feats, preferred_element_type=jnp.float32)
    deg = jnp.sum(adj, axis=1, keepdims=True) + 1.0
    z = jnp.dot(feats_blk, wl1.T, preferred_element_type=jnp.float32)
    return z + jnp.dot(acc, wc, preferred_element_type=jnp.float32) / deg


def _sage_block(adj_a_ref, adj_b_ref, feats_ref, fa_ref, fb_ref, wn_ref, wl_ref,
                out_a_ref, out_b_ref, *, d):
    feats = feats_ref[...]
    wl = wl_ref[...]
    wl1 = wl[:, :d]
    wc = jnp.dot(wn_ref[...].T, wl[:, d:].T, preferred_element_type=jnp.float32)
    out_a_ref[0] = _half(adj_a_ref[0], feats, fa_ref[0], wl1, wc)
    out_b_ref[0] = _half(adj_b_ref[0], feats, fb_ref[0], wl1, wc)


@jax.jit
def kernel(features, adj, W_neigh, W_lin):
    n, d = features.shape
    out = W_lin.shape[0]
    bm = 200
    nblk = n // bm
    grid = (nblk // 2,)
    adj3 = adj.reshape(nblk, bm, n)
    feats3 = features.reshape(nblk, bm, d)
    out_a, out_b = pl.pallas_call(
        functools.partial(_sage_block, d=d),
        grid=grid,
        in_specs=[
            pl.BlockSpec((1, bm, n), lambda i: (2 * i, 0, 0)),
            pl.BlockSpec((1, bm, n), lambda i: (2 * i + 1, 0, 0)),
            pl.BlockSpec((n, d), lambda i: (0, 0)),
            pl.BlockSpec((1, bm, d), lambda i: (2 * i, 0, 0)),
            pl.BlockSpec((1, bm, d), lambda i: (2 * i + 1, 0, 0)),
            pl.BlockSpec((d, d), lambda i: (0, 0)),
            pl.BlockSpec((out, 2 * d), lambda i: (0, 0)),
        ],
        out_specs=[
            pl.BlockSpec((1, bm, out), lambda i: (i, 0, 0)),
            pl.BlockSpec((1, bm, out), lambda i: (i, 0, 0)),
        ],
        out_shape=[
            jax.ShapeDtypeStruct((grid[0], bm, out), jnp.float32),
            jax.ShapeDtypeStruct((grid[0], bm, out), jnp.float32),
        ],
        compiler_params=pltpu.CompilerParams(
            dimension_semantics=("arbitrary",),
        ),
    )(adj3, adj3, features, feats3, feats3, W_neigh, W_lin)
    z = jnp.stack([out_a, out_b], axis=1)
    return z.reshape(n, out)


# speedup vs baseline: 1.1418x; 1.1418x over previous
import functools

import jax
import jax.numpy as jnp
from jax.experimental import pallas as pl
from jax.experimental.pallas import tpu as pltpu


def _sage_block(adj_ref, feats_ref, wn_ref, wl_ref, out_ref, *, d, bm):
    adj = adj_ref[...]
    feats = feats_ref[...]
    acc = jnp.dot(adj, feats, preferred_element_type=jnp.float32)
    deg = jnp.sum(adj, axis=1, keepdims=True) + 1.0
    wl = wl_ref[...]
    wl1 = wl[:, :d]
    wc = jnp.dot(wn_ref[...].T, wl[:, d:].T, preferred_element_type=jnp.float32)
    feats_blk = feats_ref[pl.ds(pl.program_id(0) * bm, bm), :]
    z = jnp.dot(feats_blk, wl1.T, preferred_element_type=jnp.float32)
    out_ref[...] = z + jnp.dot(acc, wc, preferred_element_type=jnp.float32) / deg


@jax.jit
def kernel(features, adj, W_neigh, W_lin):
    n, d = features.shape
    out = W_lin.shape[0]
    bm = 400
    grid = (n // bm,)
    return pl.pallas_call(
        functools.partial(_sage_block, d=d, bm=bm),
        grid=grid,
        in_specs=[
            pl.BlockSpec((bm, n), lambda i: (i, 0)),
            pl.BlockSpec((n, d), lambda i: (0, 0)),
            pl.BlockSpec((d, d), lambda i: (0, 0)),
            pl.BlockSpec((out, 2 * d), lambda i: (0, 0)),
        ],
        out_specs=pl.BlockSpec((bm, out), lambda i: (i, 0)),
        out_shape=jax.ShapeDtypeStruct((n, out), jnp.float32),
        compiler_params=pltpu.CompilerParams(
            dimension_semantics=("arbitrary",),
        ),
    )(adj, features, W_neigh, W_lin)


# speedup vs baseline: 1.1583x; 1.0145x over previous
import functools

import jax
import jax.numpy as jnp
from jax.experimental import pallas as pl
from jax.experimental.pallas import tpu as pltpu


def _sage_block(adj_ref, feats_ref, wn_ref, wl_ref, out_ref, *, d, bm):
    adj = adj_ref[...]
    feats = feats_ref[...]
    acc = jnp.dot(adj, feats, preferred_element_type=jnp.float32)
    deg = jnp.sum(adj, axis=1, keepdims=True) + 1.0
    wl = wl_ref[...]
    wl1 = wl[:, :d]
    wc = jnp.dot(wn_ref[...].T, wl[:, d:].T, preferred_element_type=jnp.float32)
    feats_blk = feats_ref[pl.ds(pl.program_id(0) * bm, bm), :]
    z = jnp.dot(feats_blk, wl1.T, preferred_element_type=jnp.float32)
    out_ref[...] = z + jnp.dot(acc, wc, preferred_element_type=jnp.float32) / deg


@jax.jit
def kernel(features, adj, W_neigh, W_lin):
    n, d = features.shape
    out = W_lin.shape[0]
    bm = 400
    grid = (n // bm,)
    return pl.pallas_call(
        functools.partial(_sage_block, d=d, bm=bm),
        grid=grid,
        in_specs=[
            pl.BlockSpec((bm, n), lambda i: (i, 0)),
            pl.BlockSpec((n, d), lambda i: (0, 0)),
            pl.BlockSpec((d, d), lambda i: (0, 0)),
            pl.BlockSpec((out, 2 * d), lambda i: (0, 0)),
        ],
        out_specs=pl.BlockSpec((bm, out), lambda i: (i, 0)),
        out_shape=jax.ShapeDtypeStruct((n, out), jnp.float32),
        compiler_params=pltpu.CompilerParams(
            dimension_semantics=("parallel",),
        ),
    )(adj, features, W_neigh, W_lin)
